# baseline (speedup 1.0000x reference)
import functools

import jax
import jax.numpy as jnp
from jax.experimental import pallas as pl
from jax.experimental.pallas import tpu as pltpu


def _fused_lowrank_body(xa_ref, xb_ref, w1a_ref, w1b_ref, w2t_ref, b2_ref,
                        o_ref):
    h = jnp.dot(xa_ref[...], w1a_ref[...], preferred_element_type=jnp.float32)
    h = h + jnp.dot(xb_ref[...], w1b_ref[...],
                    preferred_element_type=jnp.float32)
    y = jnp.dot(h.astype(w2t_ref.dtype), w2t_ref[...],
                preferred_element_type=jnp.float32)
    o_ref[...] = (y + b2_ref[...]).astype(o_ref.dtype)


@functools.partial(jax.jit, static_argnames=("tm",))
def _lowrank_call(x, w1t, w2t, b2p, tm):
    B, d_in = x.shape
    rank_p = w1t.shape[1]
    d_out_p = w2t.shape[1]
    grid = pl.cdiv(B, tm)
    d2 = d_in // 2
    return pl.pallas_call(
        _fused_lowrank_body,
        out_shape=jax.ShapeDtypeStruct((B, d_out_p), jnp.bfloat16),
        grid=(grid,),
        in_specs=[
            pl.BlockSpec((tm, d2), lambda i: (i, 0)),
            pl.BlockSpec((tm, d2), lambda i: (i, 1)),
            pl.BlockSpec((d2, rank_p), lambda i: (0, 0)),
            pl.BlockSpec((d2, rank_p), lambda i: (1, 0)),
            pl.BlockSpec((rank_p, d_out_p), lambda i: (0, 0)),
            pl.BlockSpec((1, d_out_p), lambda i: (0, 0)),
        ],
        out_specs=pl.BlockSpec((tm, d_out_p), lambda i: (i, 0)),
        compiler_params=pltpu.CompilerParams(
            dimension_semantics=("parallel",),
            vmem_limit_bytes=100 * 1024 * 1024,
        ),
    )(x, x, w1t, w1t, w2t, b2p)


def kernel(x, w1t, w2t, b2p):
    B = x.shape[0]
    tm = 512
    while tm > 8 and B % tm:
        tm //= 2
    x = x if x.dtype == w1t.dtype else x.astype(w1t.dtype)
    return _lowrank_call(x, w1t, w2t, b2p, max(tm, 8))


# speedup vs baseline: 1.2002x; 1.2002x over previous
import functools

import jax
import jax.numpy as jnp
from jax.experimental import pallas as pl
from jax.experimental.pallas import tpu as pltpu


def _fused_lowrank_body(xa_ref, xb_ref, w1a_ref, w1b_ref, w2t_ref, b2_ref,
                        o_ref):
    h = jnp.dot(xa_ref[...], w1a_ref[...], preferred_element_type=jnp.float32)
    h = h + jnp.dot(xb_ref[...], w1b_ref[...],
                    preferred_element_type=jnp.float32)
    y = jnp.dot(h.astype(w2t_ref.dtype), w2t_ref[...],
                preferred_element_type=jnp.float32)
    o_ref[...] = (y + b2_ref[...]).astype(o_ref.dtype)


@functools.partial(jax.jit, static_argnames=("tm",))
def _lowrank_call(x, w1t, w2t, b2p, tm):
    B, d_in = x.shape
    rank_p = w1t.shape[1]
    d_out_p = w2t.shape[1]
    grid = pl.cdiv(B, tm)
    d2 = d_in // 2
    return pl.pallas_call(
        _fused_lowrank_body,
        out_shape=jax.ShapeDtypeStruct((B, d_out_p), jnp.bfloat16),
        grid=(grid,),
        in_specs=[
            pl.BlockSpec((tm, d2), lambda i: (i, 0)),
            pl.BlockSpec((tm, d2), lambda i: (i, 1)),
            pl.BlockSpec((d2, rank_p), lambda i: (0, 0)),
            pl.BlockSpec((d2, rank_p), lambda i: (1, 0)),
            pl.BlockSpec((rank_p, d_out_p), lambda i: (0, 0)),
            pl.BlockSpec((1, d_out_p), lambda i: (0, 0)),
        ],
        out_specs=pl.BlockSpec((tm, d_out_p), lambda i: (i, 0)),
        compiler_params=pltpu.CompilerParams(
            dimension_semantics=("parallel",),
            vmem_limit_bytes=100 * 1024 * 1024,
        ),
    )(x, x, w1t, w1t, w2t, b2p)


def kernel(x, w1t, w2t, b2p):
    B = x.shape[0]
    tm = 256
    while tm > 8 and B % tm:
        tm //= 2
    x = x if x.dtype == w1t.dtype else x.astype(w1t.dtype)
    return _lowrank_call(x, w1t, w2t, b2p, max(tm, 8))
